# baseline (speedup 1.0000x reference)
import functools

import jax
import jax.numpy as jnp
from jax import lax
from jax.experimental import pallas as pl
from jax.experimental.pallas import tpu as pltpu
from jax.experimental.pallas import tpu_sc as plsc

_BZ = 4096
_K = 512
_D = 64
_ROWS_BLK = 4096
_N_BLK = _BZ // _ROWS_BLK

_NC = 1
_NS = 16
_NW = _NC * _NS
_B_PER_W = _BZ // _NW
_HALF = _B_PER_W // 2


def _argmin_body(xt_ref, embt_ref, idx_ref):
    xtb = xt_ref[...]
    et = embt_ref[...]
    en = jnp.sum(et * et, axis=0, keepdims=True)
    ea = jnp.concatenate([et, en], axis=0)
    xa = jnp.concatenate(
        [xtb * -2.0, jnp.full((1, _ROWS_BLK), 1.0, dtype=jnp.float32)], axis=0
    )
    dist = lax.dot_general(
        ea, xa, (((0,), (0,)), ((), ())),
        preferred_element_type=jnp.float32,
        precision=lax.Precision.HIGHEST,
    )
    m = jnp.min(dist, axis=0, keepdims=True)
    row = lax.broadcasted_iota(jnp.int32, dist.shape, 0)
    idx = jnp.min(jnp.where(dist == m, row, _K), axis=0)
    idx_ref[...] = idx


_argmin_call = pl.pallas_call(
    _argmin_body,
    grid=(_N_BLK,),
    in_specs=[
        pl.BlockSpec((_D, _ROWS_BLK), lambda i: (0, i)),
        pl.BlockSpec((_D, _K), lambda i: (0, 0)),
    ],
    out_specs=pl.BlockSpec((_ROWS_BLK,), lambda i: (i,)),
    out_shape=jax.ShapeDtypeStruct((_BZ,), jnp.int32),
)


@functools.cache
def _make_sc_gather():
    @functools.partial(
        pl.kernel,
        out_type=jax.ShapeDtypeStruct((_BZ, _D), jnp.float32),
        mesh=plsc.VectorSubcoreMesh(
            core_axis_name="c", subcore_axis_name="s",
            num_cores=_NC, num_subcores=_NS,
        ),
        scratch_types=[
            pltpu.VMEM((_B_PER_W,), jnp.int32),
            pltpu.VMEM((_B_PER_W, _D), jnp.float32),
            pltpu.SemaphoreType.DMA,
        ],
        compiler_params=pltpu.CompilerParams(use_tc_tiling_on_sc=False),
    )
    def _sc_gather(emb_hbm, idx_hbm, out_hbm, idx_v, rows_v, sem):
        wid = lax.axis_index("s") * _NC + lax.axis_index("c")
        base = wid * _B_PER_W
        pltpu.sync_copy(idx_hbm.at[pl.ds(base, _B_PER_W)], idx_v)
        pltpu.async_copy(emb_hbm.at[idx_v], rows_v, sem).wait()
        pltpu.sync_copy(rows_v, out_hbm.at[pl.ds(base, _B_PER_W)])

    return _sc_gather


def kernel(x, emb):
    idx = _argmin_call(x.T, emb.T)
    return _make_sc_gather()(emb, idx)


# speedup vs baseline: 1.0104x; 1.0104x over previous
import functools

import jax
import jax.numpy as jnp
from jax import lax
from jax.experimental import pallas as pl
from jax.experimental.pallas import tpu as pltpu
from jax.experimental.pallas import tpu_sc as plsc

_BZ = 4096
_K = 512
_D = 64
_ROWS_BLK = 4096
_N_BLK = _BZ // _ROWS_BLK

_NC = 1
_NS = 16
_NW = _NC * _NS
_B_PER_W = _BZ // _NW
_HALF = _B_PER_W // 2


def _argmin_body(xt_ref, emb_ref, idx_ref):
    xtb = xt_ref[...]
    e = emb_ref[...]
    scores = lax.dot_general(
        e, xtb, (((1,), (0,)), ((), ())),
        preferred_element_type=jnp.float32,
        precision=lax.Precision.HIGHEST,
    )
    en = jnp.sum(e * e, axis=1, keepdims=True)
    dist = en - 2.0 * scores
    m = jnp.min(dist, axis=0, keepdims=True)
    row = lax.broadcasted_iota(jnp.int32, dist.shape, 0)
    idx = jnp.min(jnp.where(dist == m, row, _K), axis=0)
    idx_ref[...] = idx


_argmin_call = pl.pallas_call(
    _argmin_body,
    grid=(_N_BLK,),
    in_specs=[
        pl.BlockSpec((_D, _ROWS_BLK), lambda i: (0, i)),
        pl.BlockSpec((_K, _D), lambda i: (0, 0)),
    ],
    out_specs=pl.BlockSpec((_ROWS_BLK,), lambda i: (i,)),
    out_shape=jax.ShapeDtypeStruct((_BZ,), jnp.int32),
)


@functools.cache
def _make_sc_gather():
    @functools.partial(
        pl.kernel,
        out_type=jax.ShapeDtypeStruct((_BZ, _D), jnp.float32),
        mesh=plsc.VectorSubcoreMesh(
            core_axis_name="c", subcore_axis_name="s",
            num_cores=_NC, num_subcores=_NS,
        ),
        scratch_types=[
            pltpu.VMEM((_B_PER_W,), jnp.int32),
            pltpu.VMEM((_B_PER_W, _D), jnp.float32),
            pltpu.SemaphoreType.DMA,
        ],
        compiler_params=pltpu.CompilerParams(use_tc_tiling_on_sc=False),
    )
    def _sc_gather(emb_hbm, idx_hbm, out_hbm, idx_v, rows_v, sem):
        wid = lax.axis_index("s") * _NC + lax.axis_index("c")
        base = wid * _B_PER_W
        pltpu.sync_copy(idx_hbm.at[pl.ds(base, _B_PER_W)], idx_v)
        pltpu.async_copy(emb_hbm.at[idx_v], rows_v, sem).wait()
        pltpu.sync_copy(rows_v, out_hbm.at[pl.ds(base, _B_PER_W)])

    return _sc_gather


def kernel(x, emb):
    idx = _argmin_call(x.T, emb)
    return _make_sc_gather()(emb, idx)


# speedup vs baseline: 1.0111x; 1.0007x over previous
import functools

import jax
import jax.numpy as jnp
from jax import lax
from jax.experimental import pallas as pl
from jax.experimental.pallas import tpu as pltpu
from jax.experimental.pallas import tpu_sc as plsc

_BZ = 4096
_K = 512
_D = 64
_ROWS_BLK = 4096
_N_BLK = _BZ // _ROWS_BLK

_NC = 1
_NS = 16
_NW = _NC * _NS
_B_PER_W = _BZ // _NW


def _argmin_body(xt_ref, emb_ref, idx_ref):
    xtb = xt_ref[...]
    e = emb_ref[...]
    scores = lax.dot_general(
        e, xtb, (((1,), (0,)), ((), ())),
        preferred_element_type=jnp.float32,
        precision=lax.Precision.HIGHEST,
    )
    en = jnp.sum(e * e, axis=1, keepdims=True)
    dist = en - 2.0 * scores
    m = jnp.min(dist, axis=0, keepdims=True)
    row = lax.broadcasted_iota(jnp.int32, dist.shape, 0)
    idx = jnp.min(jnp.where(dist == m, row, _K), axis=0)
    idx_ref[...] = idx


_argmin_call = pl.pallas_call(
    _argmin_body,
    grid=(_N_BLK,),
    in_specs=[
        pl.BlockSpec((_D, _ROWS_BLK), lambda i: (0, i)),
        pl.BlockSpec((_K, _D), lambda i: (0, 0)),
    ],
    out_specs=pl.BlockSpec((_ROWS_BLK,), lambda i: (i,)),
    out_shape=jax.ShapeDtypeStruct((_BZ,), jnp.int32),
)


@functools.cache
def _make_sc_gather():
    @functools.partial(
        pl.kernel,
        out_type=jax.ShapeDtypeStruct((_BZ, _D), jnp.float32),
        mesh=plsc.VectorSubcoreMesh(
            core_axis_name="c", subcore_axis_name="s",
            num_cores=_NC, num_subcores=_NS,
        ),
        scratch_types=[
            pltpu.VMEM((_B_PER_W,), jnp.int32),
            pltpu.VMEM((_B_PER_W, _D), jnp.float32),
            pltpu.SemaphoreType.DMA,
        ],
        compiler_params=pltpu.CompilerParams(use_tc_tiling_on_sc=False),
    )
    def _sc_gather(emb_hbm, idx_hbm, out_hbm, idx_v, rows_v, sem):
        wid = lax.axis_index("s") * _NC + lax.axis_index("c")
        base = wid * _B_PER_W
        pltpu.sync_copy(idx_hbm.at[pl.ds(base, _B_PER_W)], idx_v)
        pltpu.async_copy(emb_hbm.at[idx_v], rows_v, sem).wait()
        pltpu.sync_copy(rows_v, out_hbm.at[pl.ds(base, _B_PER_W)])

    return _sc_gather


def kernel(x, emb):
    idx = _argmin_call(x.T, emb)
    return _make_sc_gather()(emb, idx)
